# Initial kernel scaffold; baseline (speedup 1.0000x reference)
import jax
import jax.numpy as jnp
from jax.experimental import pallas as pl


def kernel(x, embed):
    raise NotImplementedError("write your pallas kernel here")


# speedup vs baseline: 1.3627x; 1.3627x over previous
import jax
import jax.numpy as jnp
from jax.experimental import pallas as pl
from jax.experimental.pallas import tpu as pltpu

_DIM = 64
_N_EMBED = 1024
_TOKENS = 1024
_N_BATCH = 16
_COMMIT = 1.0


def _vq_body(x_ref, embed_ref, q_ref, diff_ref, counts_ref, perp_ref):
    n = pl.program_id(0)
    xb = x_ref[0]
    embed = embed_ref[...]

    e2 = jnp.sum(embed * embed, axis=0, keepdims=True)
    x2 = jnp.sum(xb * xb, axis=0)
    mm = jax.lax.dot_general(xb, embed, (((0,), (0,)), ((), ())),
                             preferred_element_type=jnp.float32)
    dist = (x2[:, None] - 2.0 * mm) + e2

    idx = jnp.argmin(dist, axis=1)
    mind = jnp.min(dist, axis=1)
    diff_ref[n, 0] = jnp.sum(mind) * (_COMMIT / (_TOKENS * _DIM))

    onehot_t = (jax.lax.broadcasted_iota(jnp.int32, (_N_EMBED, _TOKENS), 0)
                == idx[None, :]).astype(jnp.float32)
    q_ref[0] = jax.lax.dot_general(
        embed, onehot_t, (((1,), (0,)), ((), ())),
        preferred_element_type=jnp.float32,
        precision=jax.lax.Precision.HIGHEST)

    cnt = jnp.sum(onehot_t, axis=1)[None, :]

    @pl.when(n == 0)
    def _init():
        counts_ref[...] = cnt

    @pl.when(n > 0)
    def _acc():
        counts_ref[...] = counts_ref[...] + cnt

    @pl.when(n == _N_BATCH - 1)
    def _finish():
        avg = counts_ref[...] * (1.0 / (_N_BATCH * _TOKENS))
        perp_ref[0, 0] = jnp.exp(-jnp.sum(avg * jnp.log(avg + 1e-10)))


def _vq_call(x3, embed, interpret=False):
    return pl.pallas_call(
        _vq_body,
        grid=(_N_BATCH,),
        in_specs=[
            pl.BlockSpec((1, _DIM, _TOKENS), lambda n: (n, 0, 0)),
            pl.BlockSpec((_DIM, _N_EMBED), lambda n: (0, 0)),
        ],
        out_specs=[
            pl.BlockSpec((1, _DIM, _TOKENS), lambda n: (n, 0, 0)),
            pl.BlockSpec((_N_BATCH, 1), lambda n: (0, 0),
                         memory_space=pltpu.SMEM),
            pl.BlockSpec((1, _N_EMBED), lambda n: (0, 0)),
            pl.BlockSpec((1, 1), lambda n: (0, 0),
                         memory_space=pltpu.SMEM),
        ],
        out_shape=[
            jax.ShapeDtypeStruct((_N_BATCH, _DIM, _TOKENS), jnp.float32),
            jax.ShapeDtypeStruct((_N_BATCH, 1), jnp.float32),
            jax.ShapeDtypeStruct((1, _N_EMBED), jnp.float32),
            jax.ShapeDtypeStruct((1, 1), jnp.float32),
        ],
        compiler_params=pltpu.CompilerParams(
            dimension_semantics=("arbitrary",)),
        interpret=interpret,
    )(x3, embed)


@jax.jit
def kernel(x, embed):
    x3 = x.reshape(_N_BATCH, _DIM, _TOKENS)
    q, diff, _counts, perp = _vq_call(x3, embed)
    return q.reshape(x.shape), diff, perp[0, 0]


# speedup vs baseline: 1.9606x; 1.4388x over previous
import jax
import jax.numpy as jnp
from jax.experimental import pallas as pl
from jax.experimental.pallas import tpu as pltpu

_DIM = 64
_N_EMBED = 1024
_TOKENS = 1024
_N_BATCH = 16
_COMMIT = 1.0


def _vq_body(x_ref, embed_ref, q_ref, diff_ref, counts_ref, perp_ref):
    n = pl.program_id(0)
    xb = x_ref[0]
    embed = embed_ref[...]

    e2 = jnp.sum(embed * embed, axis=0, keepdims=True)
    x2 = jnp.sum(xb * xb, axis=0)
    mm2 = jax.lax.dot_general(xb, -2.0 * embed, (((0,), (0,)), ((), ())),
                              preferred_element_type=jnp.float32)
    dist = (x2[:, None] + mm2) + e2

    idx = jnp.argmin(dist, axis=1)

    onehot_t = (jax.lax.broadcasted_iota(jnp.int32, (_N_EMBED, _TOKENS), 0)
                == idx[None, :]).astype(jnp.float32)
    q = jax.lax.dot_general(embed, onehot_t, (((1,), (0,)), ((), ())),
                            preferred_element_type=jnp.float32)
    q_ref[0] = q

    r = q - xb
    diff_ref[n, 0] = jnp.sum(r * r) * (_COMMIT / (_TOKENS * _DIM))

    cnt = jax.lax.dot_general(
        onehot_t, jnp.ones((_TOKENS, 8), jnp.float32),
        (((1,), (0,)), ((), ())),
        preferred_element_type=jnp.float32)

    @pl.when(n == 0)
    def _init():
        counts_ref[...] = cnt

    @pl.when(n > 0)
    def _acc():
        counts_ref[...] = counts_ref[...] + cnt

    @pl.when(n == _N_BATCH - 1)
    def _finish():
        avg = counts_ref[:, 0:1] * (1.0 / (_N_BATCH * _TOKENS))
        perp_ref[0, 0] = jnp.exp(-jnp.sum(avg * jnp.log(avg + 1e-10)))


def _vq_call(x3, embed, interpret=False):
    return pl.pallas_call(
        _vq_body,
        grid=(_N_BATCH,),
        in_specs=[
            pl.BlockSpec((1, _DIM, _TOKENS), lambda n: (n, 0, 0)),
            pl.BlockSpec((_DIM, _N_EMBED), lambda n: (0, 0)),
        ],
        out_specs=[
            pl.BlockSpec((1, _DIM, _TOKENS), lambda n: (n, 0, 0)),
            pl.BlockSpec((_N_BATCH, 1), lambda n: (0, 0),
                         memory_space=pltpu.SMEM),
            pl.BlockSpec((_N_EMBED, 8), lambda n: (0, 0)),
            pl.BlockSpec((1, 1), lambda n: (0, 0),
                         memory_space=pltpu.SMEM),
        ],
        out_shape=[
            jax.ShapeDtypeStruct((_N_BATCH, _DIM, _TOKENS), jnp.float32),
            jax.ShapeDtypeStruct((_N_BATCH, 1), jnp.float32),
            jax.ShapeDtypeStruct((_N_EMBED, 8), jnp.float32),
            jax.ShapeDtypeStruct((1, 1), jnp.float32),
        ],
        compiler_params=pltpu.CompilerParams(
            dimension_semantics=("arbitrary",)),
        interpret=interpret,
    )(x3, embed)


@jax.jit
def kernel(x, embed):
    x3 = x.reshape(_N_BATCH, _DIM, _TOKENS)
    q, diff, _counts, perp = _vq_call(x3, embed)
    return q.reshape(x.shape), diff, perp[0, 0]


# speedup vs baseline: 2.4766x; 1.2631x over previous
import jax
import jax.numpy as jnp
from jax.experimental import pallas as pl
from jax.experimental.pallas import tpu as pltpu

_DIM = 64
_N_EMBED = 1024
_TOKENS = 1024
_N_BATCH = 16
_COMMIT = 1.0


_BPS = 2
_STEPS = _N_BATCH // _BPS


def _vq_body(x_ref, embed_ref, q_ref, diff_ref, counts_ref, perp_ref):
    n = pl.program_id(0)
    embed = embed_ref[...]

    e2c = jnp.sum(embed * embed, axis=0)[:, None]
    nembed2 = -2.0 * embed

    cnt = None
    for j in range(_BPS):
        xb = x_ref[j]
        x2 = jnp.sum(xb * xb, axis=0)
        mm2 = jax.lax.dot_general(nembed2, xb, (((0,), (0,)), ((), ())),
                                  preferred_element_type=jnp.float32)
        dist = (x2[None, :] + mm2) + e2c

        idx = jnp.argmin(dist, axis=0)

        onehot_t = (jax.lax.broadcasted_iota(
            jnp.int32, (_N_EMBED, _TOKENS), 0)
            == idx[None, :]).astype(jnp.float32)
        q = jax.lax.dot_general(embed, onehot_t, (((1,), (0,)), ((), ())),
                                preferred_element_type=jnp.float32)
        q_ref[j] = q

        r = q - xb
        diff_ref[n * _BPS + j, 0] = (
            jnp.sum(r * r) * (_COMMIT / (_TOKENS * _DIM)))

        c = jax.lax.dot_general(
            onehot_t, jnp.ones((_TOKENS, 8), jnp.float32),
            (((1,), (0,)), ((), ())),
            preferred_element_type=jnp.float32)
        cnt = c if cnt is None else cnt + c

    @pl.when(n == 0)
    def _init():
        counts_ref[...] = cnt

    @pl.when(n > 0)
    def _acc():
        counts_ref[...] = counts_ref[...] + cnt

    @pl.when(n == _STEPS - 1)
    def _finish():
        avg = counts_ref[:, 0:1] * (1.0 / (_N_BATCH * _TOKENS))
        perp_ref[0, 0] = jnp.exp(-jnp.sum(avg * jnp.log(avg + 1e-10)))


def _vq_call(x3, embed, interpret=False):
    return pl.pallas_call(
        _vq_body,
        grid=(_STEPS,),
        in_specs=[
            pl.BlockSpec((_BPS, _DIM, _TOKENS), lambda n: (n, 0, 0)),
            pl.BlockSpec((_DIM, _N_EMBED), lambda n: (0, 0)),
        ],
        out_specs=[
            pl.BlockSpec((_BPS, _DIM, _TOKENS), lambda n: (n, 0, 0)),
            pl.BlockSpec((_N_BATCH, 1), lambda n: (0, 0),
                         memory_space=pltpu.SMEM),
            pl.BlockSpec((_N_EMBED, 8), lambda n: (0, 0)),
            pl.BlockSpec((1, 1), lambda n: (0, 0),
                         memory_space=pltpu.SMEM),
        ],
        out_shape=[
            jax.ShapeDtypeStruct((_N_BATCH, _DIM, _TOKENS), jnp.float32),
            jax.ShapeDtypeStruct((_N_BATCH, 1), jnp.float32),
            jax.ShapeDtypeStruct((_N_EMBED, 8), jnp.float32),
            jax.ShapeDtypeStruct((1, 1), jnp.float32),
        ],
        compiler_params=pltpu.CompilerParams(
            dimension_semantics=("arbitrary",)),
        interpret=interpret,
    )(x3, embed)


@jax.jit
def kernel(x, embed):
    x3 = x.reshape(_N_BATCH, _DIM, _TOKENS)
    q, diff, _counts, perp = _vq_call(x3, embed)
    return q.reshape(x.shape), diff, perp[0, 0]
